# Initial kernel scaffold; baseline (speedup 1.0000x reference)
import jax
import jax.numpy as jnp
from jax.experimental import pallas as pl


def kernel(latents, node_features, senders, receivers, edge_features, W_t2g, b_t2g, W_edge, b_edge, W_gnn, b_gnn, W_dec, b_dec, W_actor, b_actor, W_critic, b_critic):
    raise NotImplementedError("write your pallas kernel here")


# speedup vs baseline: 3.2107x; 3.2107x over previous
import functools

import jax
import jax.numpy as jnp
from jax import lax
from jax.experimental import pallas as pl
from jax.experimental.pallas import tpu as pltpu
from jax.experimental.pallas import tpu_sc as plsc

N = 10000
E = 320000
P = 1024
DF = 128
DG = 128
DE = 16
A = 8

NC = 2
NS = 16
NW = NC * NS
NE_TILE = E // NW
CH = 80
NCH = NE_TILE // CH
NP_ = 10240
ROWS_TILE = NP_ // NS
ZROWS = 128

_f32 = jnp.float32


def _sc_step_body(nf_hbm, s_hbm, r_hbm, ee_hbm, out_hbm,
                  idx_v, ridx_v, rows_v, ee_v, zbuf, agg_sh, sem):
    cid = lax.axis_index("c")
    sid = lax.axis_index("s")
    wid = cid * NS + sid

    def _zero_row(r, _):
        for j in range(DG // 16):
            zbuf[r, pl.ds(j * 16, 16)] = jnp.zeros((16,), _f32)
        return 0
    lax.fori_loop(0, ZROWS, _zero_row, 0)
    for j in range(ROWS_TILE // ZROWS):
        pltpu.sync_copy(zbuf, agg_sh.at[pl.ds(sid * ROWS_TILE + j * ZROWS, ZROWS)])
    plsc.subcore_barrier()

    def _chunk(k, _):
        base = wid * NE_TILE + k * CH
        pltpu.sync_copy(s_hbm.at[pl.ds(base, CH)], idx_v)
        gcp = pltpu.async_copy(nf_hbm.at[idx_v], rows_v, sem)
        pltpu.sync_copy(ee_hbm.at[pl.ds(base, CH)], ee_v)
        pltpu.sync_copy(r_hbm.at[pl.ds(base, CH)], ridx_v)
        gcp.wait()

        def _row(r, _):
            for j in range(DG // 16):
                c = j * 16
                v = rows_v[r, pl.ds(c, 16)] + ee_v[r, pl.ds(c, 16)]
                rows_v[r, pl.ds(c, 16)] = jnp.maximum(v, 0.0)
            return 0
        lax.fori_loop(0, CH, _row, 0)

        pltpu.sync_copy(rows_v, agg_sh.at[ridx_v], add=True)
        return 0
    lax.fori_loop(0, NCH, _chunk, 0)
    plsc.subcore_barrier()

    pltpu.sync_copy(agg_sh.at[pl.ds(sid * ROWS_TILE, ROWS_TILE)],
                    out_hbm.at[cid, pl.ds(sid * ROWS_TILE, ROWS_TILE)])


_sc_step = pl.kernel(
    _sc_step_body,
    out_type=jax.ShapeDtypeStruct((NC, NP_, DG), _f32),
    mesh=plsc.VectorSubcoreMesh(core_axis_name="c", subcore_axis_name="s",
                                num_cores=NC, num_subcores=NS),
    scratch_types=[
        pltpu.VMEM((CH,), jnp.int32),
        pltpu.VMEM((CH,), jnp.int32),
        pltpu.VMEM((CH, DG), _f32),
        pltpu.VMEM((CH, DG), _f32),
        pltpu.VMEM((ZROWS, DG), _f32),
        pltpu.VMEM_SHARED((NP_, DG), _f32),
        pltpu.SemaphoreType.DMA,
    ],
)


def _nf_body(x_ref, w_ref, b_ref, o_ref):
    o_ref[...] = jnp.dot(x_ref[...], w_ref[...],
                         preferred_element_type=_f32) + b_ref[...]


def _node_encode(x, w, b2):
    return pl.pallas_call(
        _nf_body,
        out_shape=jax.ShapeDtypeStruct((N, DG), _f32),
    )(x, w, b2)


_EB = 8000


def _edge_proj(ef, w, b2):
    return pl.pallas_call(
        _nf_body,
        grid=(E // _EB,),
        in_specs=[
            pl.BlockSpec((_EB, DE), lambda i: (i, 0)),
            pl.BlockSpec((DE, DG), lambda i: (0, 0)),
            pl.BlockSpec((1, DG), lambda i: (0, 0)),
        ],
        out_specs=pl.BlockSpec((_EB, DG), lambda i: (i, 0)),
        out_shape=jax.ShapeDtypeStruct((E, DG), _f32),
    )(ef, w, b2)


def _mid_body(aggs_ref, nf_ref, w_ref, b_ref, o_ref):
    a = aggs_ref[0] + aggs_ref[1]
    h = jnp.maximum(jnp.dot(a, w_ref[...], preferred_element_type=_f32)
                    + b_ref[...], 0.0)
    o_ref[...] = h + nf_ref[...]


def _mid(aggs, nf, w, b2):
    return pl.pallas_call(
        _mid_body,
        grid=(1,),
        in_specs=[
            pl.BlockSpec((NC, N, DG), lambda i: (0, 0, 0)),
            pl.BlockSpec((N, DG), lambda i: (0, 0)),
            pl.BlockSpec((DG, DG), lambda i: (0, 0)),
            pl.BlockSpec((1, DG), lambda i: (0, 0)),
        ],
        out_specs=pl.BlockSpec((N, DG), lambda i: (0, 0)),
        out_shape=jax.ShapeDtypeStruct((N, DG), _f32),
    )(aggs, nf, w, b2)


def _head_body(q_ref, lat_ref, wg_ref, bg_ref, wd_ref, bd_ref, wh_ref, bh_ref,
               o_ref):
    a = q_ref[0] + q_ref[1]
    l2 = jnp.maximum(jnp.dot(a, wg_ref[...], preferred_element_type=_f32)
                     + bg_ref[...], 0.0)
    dcd = jnp.dot(l2, wd_ref[...], preferred_element_type=_f32) + bd_ref[...]
    cat = jnp.concatenate([lat_ref[...], dcd], axis=-1)
    o_ref[...] = jnp.dot(cat, wh_ref[...], preferred_element_type=_f32) \
        + bh_ref[...]


def _heads(aggs, latents, wg, bg2, wd, bd2, wh, bh2):
    return pl.pallas_call(
        _head_body,
        grid=(1,),
        in_specs=[
            pl.BlockSpec((NC, P, DG), lambda i: (0, 0, 0)),
            pl.BlockSpec((P, DF), lambda i: (0, 0)),
            pl.BlockSpec((DG, DG), lambda i: (0, 0)),
            pl.BlockSpec((1, DG), lambda i: (0, 0)),
            pl.BlockSpec((DG, DG), lambda i: (0, 0)),
            pl.BlockSpec((1, DG), lambda i: (0, 0)),
            pl.BlockSpec((DF + DG, 128), lambda i: (0, 0)),
            pl.BlockSpec((1, 128), lambda i: (0, 0)),
        ],
        out_specs=pl.BlockSpec((P, 128), lambda i: (0, 0)),
        out_shape=jax.ShapeDtypeStruct((P, 128), _f32),
    )(aggs, latents, wg, bg2, wd, bd2, wh, bh2)


def kernel(latents, node_features, senders, receivers, edge_features,
           W_t2g, b_t2g, W_edge, b_edge, W_gnn, b_gnn, W_dec, b_dec,
           W_actor, b_actor, W_critic, b_critic):
    bg2 = b_gnn.reshape(1, DG)
    nf = _node_encode(node_features, W_t2g, b_t2g.reshape(1, DG))
    ee = _edge_proj(edge_features, W_edge, b_edge.reshape(1, DG))

    aggs1 = _sc_step(nf, senders, receivers, ee)
    nf2 = _mid(aggs1, nf, W_gnn, bg2)
    aggs2 = _sc_step(nf2, senders, receivers, ee)

    wh = jnp.zeros((DF + DG, 128), _f32)
    wh = wh.at[:, :A].set(W_actor).at[:, A:A + 1].set(W_critic)
    bh = jnp.zeros((1, 128), _f32)
    bh = bh.at[0, :A].set(b_actor).at[0, A:A + 1].set(b_critic)

    heads = _heads(aggs2, latents, W_gnn, bg2, W_dec, b_dec.reshape(1, DG),
                   wh, bh)
    value = heads[:, A:A + 1]
    policy = heads[:, :A]
    return value, policy
